# Initial kernel scaffold; baseline (speedup 1.0000x reference)
import jax
import jax.numpy as jnp
from jax.experimental import pallas as pl


def kernel(x, edge_index, edge_attr, params):
    raise NotImplementedError("write your pallas kernel here")


# speedup vs baseline: 3.6032x; 3.6032x over previous
import functools

import jax
import jax.numpy as jnp
from jax import lax
from jax.experimental import pallas as pl
from jax.experimental.pallas import tpu as pltpu
from jax.experimental.pallas import tpu_sc as plsc

_N = 10000
_E = 320000
_H = 128
_DN = 128
_DE = 16

_NSUB = 16
_NC = 2
_NW = _NSUB * _NC
_CHUNK = 128
_NCHUNK = _E // _CHUNK
_CPW = -(-_NCHUNK // _NW)
_NPAD = 10240
_RPS = _NPAD // _NSUB

_BN = 1000
_BE = 2000


def _elu(x):
    return jnp.where(x > 0, x, jnp.exp(jnp.minimum(x, 0.0)) - 1.0)


def _ln(t, g, b):
    m = jnp.mean(t, axis=-1, keepdims=True)
    v = jnp.mean((t - m) * (t - m), axis=-1, keepdims=True)
    return (t - m) * lax.rsqrt(v + 1e-5) * g + b


def _dot(a, b):
    return jnp.dot(a, b, preferred_element_type=jnp.float32)


def _w_spec(shape):
    nd = len(shape)
    return pl.BlockSpec(shape, lambda i: (0,) * nd)


def _row_spec(blk, d):
    return pl.BlockSpec((blk, d), lambda i: (i, 0))


def _node_enc_body(x_ref, w1t, b1, w2t, b2, g, bl, wst, wdt,
                   h_ref, u_ref, v_ref):
    t = _elu(_dot(x_ref[...], w1t[...]) + b1[...])
    t = _dot(t, w2t[...]) + b2[...]
    h = _ln(t, g[...], bl[...])
    h_ref[...] = h
    u_ref[...] = _dot(h, wst[...])
    v_ref[...] = _dot(h, wdt[...])


def _edge_enc_body(a_ref, w1t, b1, w2t, b2, g, bl, o_ref):
    t = _elu(_dot(a_ref[...], w1t[...]) + b1[...])
    t = _dot(t, w2t[...]) + b2[...]
    o_ref[...] = _ln(t, g[...], bl[...])


def _edge_mp_body(gu_ref, gv_ref, e_ref, wet, b1, w2t, b2, g, bl, o_ref):
    t = gu_ref[...] + gv_ref[...] + _dot(e_ref[...], wet[...]) + b1[...]
    t = _elu(t)
    t = _dot(t, w2t[...]) + b2[...]
    o_ref[...] = _ln(t, g[...], bl[...])


def _node_mp_mid_body(h_ref, s_ref, c_ref, wht, wat, b1, w2t, b2, g, bl,
                      wst, wdt, h_o, u_o, v_o):
    s = s_ref[0] + s_ref[1]
    cnt = c_ref[0, :, 0:1] + c_ref[1, :, 0:1]
    agg = s / jnp.maximum(cnt, 1.0)
    t = _elu(_dot(h_ref[...], wht[...]) + _dot(agg, wat[...]) + b1[...])
    t = _dot(t, w2t[...]) + b2[...]
    h = _ln(t, g[...], bl[...])
    h_o[...] = h
    u_o[...] = _dot(h, wst[...])
    v_o[...] = _dot(h, wdt[...])


def _node_mp_dec_body(h_ref, s_ref, c_ref, wht, wat, b1, w2t, b2, g, bl,
                      d1t, db1, d2t, db2, o_ref):
    s = s_ref[0] + s_ref[1]
    cnt = c_ref[0, :, 0:1] + c_ref[1, :, 0:1]
    agg = s / jnp.maximum(cnt, 1.0)
    t = _elu(_dot(h_ref[...], wht[...]) + _dot(agg, wat[...]) + b1[...])
    t = _dot(t, w2t[...]) + b2[...]
    h = _ln(t, g[...], bl[...])
    t = _elu(_dot(h, d1t[...]) + db1[...])
    o_ref[...] = _dot(t, d2t[...]) + db2[...]


def _sc_gather(u, v, src2, dst2):
    mesh = plsc.VectorSubcoreMesh(core_axis_name="c", subcore_axis_name="s")

    @functools.partial(
        pl.kernel,
        out_type=(jax.ShapeDtypeStruct((_E, _H), jnp.float32),
                  jax.ShapeDtypeStruct((_E, _H), jnp.float32)),
        mesh=mesh)
    def k(u_hbm, v_hbm, si_hbm, di_hbm, gu_hbm, gv_hbm):
        def body(si_v, di_v, gu_v, gv_v):
            pltpu.sync_copy(u_hbm.at[si_v.at[0]], gu_v)
            pltpu.sync_copy(v_hbm.at[di_v.at[0]], gv_v)

        pltpu.emit_pipeline(
            body,
            grid=(_NCHUNK,),
            in_specs=[pl.BlockSpec((1, _CHUNK), lambda i: (0, i)),
                      pl.BlockSpec((1, _CHUNK), lambda i: (0, i))],
            out_specs=[pl.BlockSpec((_CHUNK, _H), lambda i: (i, 0)),
                       pl.BlockSpec((_CHUNK, _H), lambda i: (i, 0))],
            core_axis_name=("c", "s"),
            dimension_semantics=(pltpu.PARALLEL,),
        )(si_hbm, di_hbm, gu_hbm, gv_hbm)

    return k(u, v, src2, dst2)


def _sc_segsum(e, dst2, z128):
    mesh = plsc.VectorSubcoreMesh(core_axis_name="c", subcore_axis_name="s")

    @functools.partial(
        pl.kernel,
        out_type=jax.ShapeDtypeStruct((_NC * _NPAD, _H), jnp.float32),
        mesh=mesh,
        scratch_types=[
            pltpu.VMEM_SHARED((_NPAD, _H), jnp.float32),
            pltpu.VMEM((_CHUNK,), jnp.int32),
            pltpu.VMEM((_CHUNK, _H), jnp.float32),
        ])
    def k(e_hbm, di_hbm, z128_hbm, sum_hbm, ssum, idx_v, rows_v):
        cid = lax.axis_index("c")
        sid = lax.axis_index("s")
        wid = sid * _NC + cid
        r0 = sid * _RPS
        pltpu.sync_copy(z128_hbm, rows_v)

        @pl.loop(0, _RPS // _CHUNK)
        def _(j):
            pltpu.sync_copy(rows_v,
                            ssum.at[pl.ds(r0 + j * _CHUNK, _CHUNK)])

        plsc.subcore_barrier()

        @pl.loop(0, _CPW)
        def _(kk):
            c = wid + kk * _NW

            @pl.when(c < _NCHUNK)
            def _():
                pltpu.sync_copy(di_hbm.at[c], idx_v)
                pltpu.sync_copy(e_hbm.at[pl.ds(c * _CHUNK, _CHUNK)],
                                rows_v)
                pltpu.sync_copy(rows_v, ssum.at[idx_v], add=True)

        plsc.subcore_barrier()
        o0 = cid * _NPAD + r0

        @pl.loop(0, _RPS // _CHUNK)
        def _(j):
            pltpu.sync_copy(ssum.at[pl.ds(r0 + j * _CHUNK, _CHUNK)],
                            rows_v)
            pltpu.sync_copy(rows_v, sum_hbm.at[pl.ds(o0 + j * _CHUNK, _CHUNK)])

    return k(e, dst2, z128)


def _sc_counts(dst2, ones128):
    mesh = plsc.VectorSubcoreMesh(core_axis_name="c", subcore_axis_name="s")

    @functools.partial(
        pl.kernel,
        out_type=jax.ShapeDtypeStruct((_NC * _NPAD, _H), jnp.float32),
        mesh=mesh,
        scratch_types=[
            pltpu.VMEM_SHARED((_NPAD, _H), jnp.float32),
            pltpu.VMEM((_CHUNK,), jnp.int32),
            pltpu.VMEM((_CHUNK, _H), jnp.float32),
        ])
    def k(di_hbm, z128_hbm, ones_hbm, cnt_hbm, scnt, idx_v, rows_v):
        cid = lax.axis_index("c")
        sid = lax.axis_index("s")
        wid = sid * _NC + cid
        r0 = sid * _RPS
        pltpu.sync_copy(z128_hbm, rows_v)

        @pl.loop(0, _RPS // _CHUNK)
        def _(j):
            pltpu.sync_copy(rows_v,
                            scnt.at[pl.ds(r0 + j * _CHUNK, _CHUNK)])

        pltpu.sync_copy(ones_hbm, rows_v)
        plsc.subcore_barrier()

        @pl.loop(0, _CPW)
        def _(kk):
            c = wid + kk * _NW

            @pl.when(c < _NCHUNK)
            def _():
                pltpu.sync_copy(di_hbm.at[c], idx_v)
                pltpu.sync_copy(rows_v, scnt.at[idx_v], add=True)

        plsc.subcore_barrier()
        o0 = cid * _NPAD + r0

        @pl.loop(0, _RPS // _CHUNK)
        def _(j):
            pltpu.sync_copy(scnt.at[pl.ds(r0 + j * _CHUNK, _CHUNK)],
                            rows_v)
            pltpu.sync_copy(rows_v, cnt_hbm.at[pl.ds(o0 + j * _CHUNK, _CHUNK)])

    return k(dst2, jnp.zeros((_CHUNK, _H), jnp.float32), ones128)


def _run_node_enc(x, ws):
    f32 = jnp.float32
    return pl.pallas_call(
        _node_enc_body,
        grid=(_N // _BN,),
        in_specs=[_row_spec(_BN, _DN)] + [_w_spec(w.shape) for w in ws],
        out_specs=[_row_spec(_BN, _H)] * 3,
        out_shape=[jax.ShapeDtypeStruct((_N, _H), f32)] * 3,
    )(x, *ws)


def _run_edge_enc(a, ws):
    f32 = jnp.float32
    return pl.pallas_call(
        _edge_enc_body,
        grid=(_E // _BE,),
        in_specs=[_row_spec(_BE, _DE)] + [_w_spec(w.shape) for w in ws],
        out_specs=_row_spec(_BE, _H),
        out_shape=jax.ShapeDtypeStruct((_E, _H), f32),
    )(a, *ws)


def _run_edge_mp(gu, gv, e, ws):
    f32 = jnp.float32
    return pl.pallas_call(
        _edge_mp_body,
        grid=(_E // _BE,),
        in_specs=[_row_spec(_BE, _H)] * 3 + [_w_spec(w.shape) for w in ws],
        out_specs=_row_spec(_BE, _H),
        out_shape=jax.ShapeDtypeStruct((_E, _H), f32),
    )(gu, gv, e, *ws)


def _node_mp_in_specs(ws):
    return ([_row_spec(_BN, _H),
             pl.BlockSpec((_NC, _BN, _H), lambda i: (0, i, 0)),
             pl.BlockSpec((_NC, _BN, _H), lambda i: (0, i, 0))]
            + [_w_spec(w.shape) for w in ws])


def _run_node_mp_mid(h, s, c, ws):
    f32 = jnp.float32
    return pl.pallas_call(
        _node_mp_mid_body,
        grid=(_N // _BN,),
        in_specs=_node_mp_in_specs(ws),
        out_specs=[_row_spec(_BN, _H)] * 3,
        out_shape=[jax.ShapeDtypeStruct((_N, _H), f32)] * 3,
    )(h, s, c, *ws)


def _run_node_mp_dec(h, s, c, ws):
    f32 = jnp.float32
    return pl.pallas_call(
        _node_mp_dec_body,
        grid=(_N // _BN,),
        in_specs=_node_mp_in_specs(ws),
        out_specs=_row_spec(_BN, _DN),
        out_shape=jax.ShapeDtypeStruct((_N, _DN), f32),
    )(h, s, c, *ws)


def kernel(x, edge_index, edge_attr, params):
    p = params
    f32 = jnp.float32
    src2 = edge_index[0].astype(jnp.int32).reshape(1, _E)
    dst = edge_index[1].astype(jnp.int32)
    dst2 = dst.reshape(_NCHUNK, _CHUNK)
    z128 = jnp.zeros((_CHUNK, _H), f32)
    ones128 = jnp.ones((_CHUNK, _H), f32)

    def lin(q):
        return q["w"].T, q["b"].reshape(1, -1)

    ne1t, ne1b = lin(p["node_enc"][0])
    ne2t, ne2b = lin(p["node_enc"][1])
    neg = p["node_enc_ln"]["g"].reshape(1, -1)
    neb = p["node_enc_ln"]["b"].reshape(1, -1)
    ee1t, ee1b = lin(p["edge_enc"][0])
    ee2t, ee2b = lin(p["edge_enc"][1])
    eeg = p["edge_enc_ln"]["g"].reshape(1, -1)
    eeb = p["edge_enc_ln"]["b"].reshape(1, -1)

    emp = []
    for i in range(2):
        w1t, b1 = lin(p["edge_mp"][i][0])
        w2t, b2 = lin(p["edge_mp"][i][1])
        emp.append(dict(
            wst=w1t[0:_H], wdt=w1t[_H:2 * _H], wet=w1t[2 * _H:3 * _H],
            b1=b1, w2t=w2t, b2=b2,
            g=p["edge_mp_ln"][i]["g"].reshape(1, -1),
            bl=p["edge_mp_ln"][i]["b"].reshape(1, -1)))

    nmp = []
    for i in range(2):
        w1t, b1 = lin(p["node_mp"][i][0])
        w2t, b2 = lin(p["node_mp"][i][1])
        nmp.append(dict(
            wht=w1t[0:_H], wat=w1t[_H:2 * _H],
            b1=b1, w2t=w2t, b2=b2,
            g=p["node_mp_ln"][i]["g"].reshape(1, -1),
            bl=p["node_mp_ln"][i]["b"].reshape(1, -1)))

    d1t, db1 = lin(p["node_dec"][0])
    d2t, db2 = lin(p["node_dec"][1])

    h, u, v = _run_node_enc(
        x, [ne1t, ne1b, ne2t, ne2b, neg, neb, emp[0]["wst"], emp[0]["wdt"]])
    e = _run_edge_enc(edge_attr, [ee1t, ee1b, ee2t, ee2b, eeg, eeb])

    cflat = _sc_counts(dst2, ones128)
    c = cflat.reshape(_NC, _NPAD, _H)[:, :_N]

    gu, gv = _sc_gather(u, v, src2, dst2.reshape(1, _E))
    m = emp[0]
    e = _run_edge_mp(gu, gv, e, [m["wet"], m["b1"], m["w2t"], m["b2"],
                                 m["g"], m["bl"]])
    sflat = _sc_segsum(e, dst2, z128)
    s = sflat.reshape(_NC, _NPAD, _H)[:, :_N]
    n = nmp[0]
    h, u, v = _run_node_mp_mid(
        h, s, c, [n["wht"], n["wat"], n["b1"], n["w2t"], n["b2"],
                  n["g"], n["bl"], emp[1]["wst"], emp[1]["wdt"]])

    gu, gv = _sc_gather(u, v, src2, dst2.reshape(1, _E))
    m = emp[1]
    e = _run_edge_mp(gu, gv, e, [m["wet"], m["b1"], m["w2t"], m["b2"],
                                 m["g"], m["bl"]])
    sflat = _sc_segsum(e, dst2, z128)
    s = sflat.reshape(_NC, _NPAD, _H)[:, :_N]
    n = nmp[1]
    out = _run_node_mp_dec(
        h, s, c, [n["wht"], n["wat"], n["b1"], n["w2t"], n["b2"],
                  n["g"], n["bl"], d1t, db1, d2t, db2])
    return out


# speedup vs baseline: 3.6649x; 1.0171x over previous
import functools

import jax
import jax.numpy as jnp
from jax import lax
from jax.experimental import pallas as pl
from jax.experimental.pallas import tpu as pltpu
from jax.experimental.pallas import tpu_sc as plsc

_N = 10000
_E = 320000
_H = 128
_DN = 128
_DE = 16

_NSUB = 16
_NC = 2
_NW = _NSUB * _NC
_CHUNK = 128
_NCHUNK = _E // _CHUNK
_CPW = -(-_NCHUNK // _NW)
_NPAD = 10240
_RPS = _NPAD // _NSUB

_BN = 1000
_BE = 2000


def _elu(x):
    return jnp.where(x > 0, x, jnp.exp(jnp.minimum(x, 0.0)) - 1.0)


def _ln(t, g, b):
    m = jnp.mean(t, axis=-1, keepdims=True)
    v = jnp.mean((t - m) * (t - m), axis=-1, keepdims=True)
    return (t - m) * lax.rsqrt(v + 1e-5) * g + b


def _dot(a, b):
    return jnp.dot(a, b, preferred_element_type=jnp.float32)


def _w_spec(shape):
    nd = len(shape)
    return pl.BlockSpec(shape, lambda i: (0,) * nd)


def _row_spec(blk, d):
    return pl.BlockSpec((blk, d), lambda i: (i, 0))


def _node_enc_body(x_ref, w1t, b1, w2t, b2, g, bl, wst, wdt,
                   h_ref, u_ref, v_ref):
    t = _elu(_dot(x_ref[...], w1t[...]) + b1[...])
    t = _dot(t, w2t[...]) + b2[...]
    h = _ln(t, g[...], bl[...])
    h_ref[...] = h
    u_ref[...] = _dot(h, wst[...])
    v_ref[...] = _dot(h, wdt[...])


def _edge_enc_body(a_ref, w1t, b1, w2t, b2, g, bl, o_ref):
    t = _elu(_dot(a_ref[...], w1t[...]) + b1[...])
    t = _dot(t, w2t[...]) + b2[...]
    o_ref[...] = _ln(t, g[...], bl[...])


def _edge_mp_body(gu_ref, gv_ref, e_ref, wet, b1, w2t, b2, g, bl, o_ref):
    t = gu_ref[...] + gv_ref[...] + _dot(e_ref[...], wet[...]) + b1[...]
    t = _elu(t)
    t = _dot(t, w2t[...]) + b2[...]
    o_ref[...] = _ln(t, g[...], bl[...])


def _node_mp_mid_body(h_ref, s_ref, c_ref, wht, wat, b1, w2t, b2, g, bl,
                      wst, wdt, h_o, u_o, v_o):
    s = s_ref[0] + s_ref[1]
    cnt = c_ref[0, :, 0:1] + c_ref[1, :, 0:1]
    agg = s / jnp.maximum(cnt, 1.0)
    t = _elu(_dot(h_ref[...], wht[...]) + _dot(agg, wat[...]) + b1[...])
    t = _dot(t, w2t[...]) + b2[...]
    h = _ln(t, g[...], bl[...])
    h_o[...] = h
    u_o[...] = _dot(h, wst[...])
    v_o[...] = _dot(h, wdt[...])


def _node_mp_dec_body(h_ref, s_ref, c_ref, wht, wat, b1, w2t, b2, g, bl,
                      d1t, db1, d2t, db2, o_ref):
    s = s_ref[0] + s_ref[1]
    cnt = c_ref[0, :, 0:1] + c_ref[1, :, 0:1]
    agg = s / jnp.maximum(cnt, 1.0)
    t = _elu(_dot(h_ref[...], wht[...]) + _dot(agg, wat[...]) + b1[...])
    t = _dot(t, w2t[...]) + b2[...]
    h = _ln(t, g[...], bl[...])
    t = _elu(_dot(h, d1t[...]) + db1[...])
    o_ref[...] = _dot(t, d2t[...]) + db2[...]


def _sc_gather(u, v, src2, dst2):
    mesh = plsc.VectorSubcoreMesh(core_axis_name="c", subcore_axis_name="s")

    @functools.partial(
        pl.kernel,
        out_type=(jax.ShapeDtypeStruct((_E, _H), jnp.float32),
                  jax.ShapeDtypeStruct((_E, _H), jnp.float32)),
        mesh=mesh,
        scratch_types=[pltpu.SemaphoreType.DMA])
    def k(u_hbm, v_hbm, si_hbm, di_hbm, gu_hbm, gv_hbm, sem):
        def body(si_v, di_v, gu_v, gv_v):
            h1 = pltpu.async_copy(u_hbm.at[si_v.at[0]], gu_v, sem)
            h2 = pltpu.async_copy(v_hbm.at[di_v.at[0]], gv_v, sem)
            h1.wait()
            h2.wait()

        pltpu.emit_pipeline(
            body,
            grid=(_NCHUNK,),
            in_specs=[pl.BlockSpec((1, _CHUNK), lambda i: (0, i)),
                      pl.BlockSpec((1, _CHUNK), lambda i: (0, i))],
            out_specs=[pl.BlockSpec((_CHUNK, _H), lambda i: (i, 0)),
                       pl.BlockSpec((_CHUNK, _H), lambda i: (i, 0))],
            core_axis_name=("c", "s"),
            dimension_semantics=(pltpu.PARALLEL,),
        )(si_hbm, di_hbm, gu_hbm, gv_hbm)

    return k(u, v, src2, dst2)


def _sc_segsum(e, dst2, z128):
    mesh = plsc.VectorSubcoreMesh(core_axis_name="c", subcore_axis_name="s")

    @functools.partial(
        pl.kernel,
        out_type=jax.ShapeDtypeStruct((_NC * _NPAD, _H), jnp.float32),
        mesh=mesh,
        scratch_types=[
            pltpu.VMEM_SHARED((_NPAD, _H), jnp.float32),
            pltpu.VMEM((_CHUNK,), jnp.int32),
            pltpu.VMEM((_CHUNK, _H), jnp.float32),
        ])
    def k(e_hbm, di_hbm, z128_hbm, sum_hbm, ssum, idx_v, rows_v):
        cid = lax.axis_index("c")
        sid = lax.axis_index("s")
        wid = sid * _NC + cid
        r0 = sid * _RPS
        pltpu.sync_copy(z128_hbm, rows_v)

        @pl.loop(0, _RPS // _CHUNK)
        def _(j):
            pltpu.sync_copy(rows_v,
                            ssum.at[pl.ds(r0 + j * _CHUNK, _CHUNK)])

        plsc.subcore_barrier()

        @pl.loop(0, _CPW)
        def _(kk):
            c = wid + kk * _NW

            @pl.when(c < _NCHUNK)
            def _():
                pltpu.sync_copy(di_hbm.at[c], idx_v)
                pltpu.sync_copy(e_hbm.at[pl.ds(c * _CHUNK, _CHUNK)],
                                rows_v)
                pltpu.sync_copy(rows_v, ssum.at[idx_v], add=True)

        plsc.subcore_barrier()
        o0 = cid * _NPAD + r0

        @pl.loop(0, _RPS // _CHUNK)
        def _(j):
            pltpu.sync_copy(ssum.at[pl.ds(r0 + j * _CHUNK, _CHUNK)],
                            rows_v)
            pltpu.sync_copy(rows_v, sum_hbm.at[pl.ds(o0 + j * _CHUNK, _CHUNK)])

    return k(e, dst2, z128)


def _sc_counts(dst2, ones128):
    mesh = plsc.VectorSubcoreMesh(core_axis_name="c", subcore_axis_name="s")

    @functools.partial(
        pl.kernel,
        out_type=jax.ShapeDtypeStruct((_NC * _NPAD, _H), jnp.float32),
        mesh=mesh,
        scratch_types=[
            pltpu.VMEM_SHARED((_NPAD, _H), jnp.float32),
            pltpu.VMEM((_CHUNK,), jnp.int32),
            pltpu.VMEM((_CHUNK, _H), jnp.float32),
        ])
    def k(di_hbm, z128_hbm, ones_hbm, cnt_hbm, scnt, idx_v, rows_v):
        cid = lax.axis_index("c")
        sid = lax.axis_index("s")
        wid = sid * _NC + cid
        r0 = sid * _RPS
        pltpu.sync_copy(z128_hbm, rows_v)

        @pl.loop(0, _RPS // _CHUNK)
        def _(j):
            pltpu.sync_copy(rows_v,
                            scnt.at[pl.ds(r0 + j * _CHUNK, _CHUNK)])

        pltpu.sync_copy(ones_hbm, rows_v)
        plsc.subcore_barrier()

        @pl.loop(0, _CPW)
        def _(kk):
            c = wid + kk * _NW

            @pl.when(c < _NCHUNK)
            def _():
                pltpu.sync_copy(di_hbm.at[c], idx_v)
                pltpu.sync_copy(rows_v, scnt.at[idx_v], add=True)

        plsc.subcore_barrier()
        o0 = cid * _NPAD + r0

        @pl.loop(0, _RPS // _CHUNK)
        def _(j):
            pltpu.sync_copy(scnt.at[pl.ds(r0 + j * _CHUNK, _CHUNK)],
                            rows_v)
            pltpu.sync_copy(rows_v, cnt_hbm.at[pl.ds(o0 + j * _CHUNK, _CHUNK)])

    return k(dst2, jnp.zeros((_CHUNK, _H), jnp.float32), ones128)


def _run_node_enc(x, ws):
    f32 = jnp.float32
    return pl.pallas_call(
        _node_enc_body,
        grid=(_N // _BN,),
        in_specs=[_row_spec(_BN, _DN)] + [_w_spec(w.shape) for w in ws],
        out_specs=[_row_spec(_BN, _H)] * 3,
        out_shape=[jax.ShapeDtypeStruct((_N, _H), f32)] * 3,
    )(x, *ws)


def _run_edge_enc(a, ws):
    f32 = jnp.float32
    return pl.pallas_call(
        _edge_enc_body,
        grid=(_E // _BE,),
        in_specs=[_row_spec(_BE, _DE)] + [_w_spec(w.shape) for w in ws],
        out_specs=_row_spec(_BE, _H),
        out_shape=jax.ShapeDtypeStruct((_E, _H), f32),
    )(a, *ws)


def _run_edge_mp(gu, gv, e, ws):
    f32 = jnp.float32
    return pl.pallas_call(
        _edge_mp_body,
        grid=(_E // _BE,),
        in_specs=[_row_spec(_BE, _H)] * 3 + [_w_spec(w.shape) for w in ws],
        out_specs=_row_spec(_BE, _H),
        out_shape=jax.ShapeDtypeStruct((_E, _H), f32),
    )(gu, gv, e, *ws)


def _node_mp_in_specs(ws):
    return ([_row_spec(_BN, _H),
             pl.BlockSpec((_NC, _BN, _H), lambda i: (0, i, 0)),
             pl.BlockSpec((_NC, _BN, _H), lambda i: (0, i, 0))]
            + [_w_spec(w.shape) for w in ws])


def _run_node_mp_mid(h, s, c, ws):
    f32 = jnp.float32
    return pl.pallas_call(
        _node_mp_mid_body,
        grid=(_N // _BN,),
        in_specs=_node_mp_in_specs(ws),
        out_specs=[_row_spec(_BN, _H)] * 3,
        out_shape=[jax.ShapeDtypeStruct((_N, _H), f32)] * 3,
    )(h, s, c, *ws)


def _run_node_mp_dec(h, s, c, ws):
    f32 = jnp.float32
    return pl.pallas_call(
        _node_mp_dec_body,
        grid=(_N // _BN,),
        in_specs=_node_mp_in_specs(ws),
        out_specs=_row_spec(_BN, _DN),
        out_shape=jax.ShapeDtypeStruct((_N, _DN), f32),
    )(h, s, c, *ws)


def kernel(x, edge_index, edge_attr, params):
    p = params
    f32 = jnp.float32
    src2 = edge_index[0].astype(jnp.int32).reshape(1, _E)
    dst = edge_index[1].astype(jnp.int32)
    dst2 = dst.reshape(_NCHUNK, _CHUNK)
    z128 = jnp.zeros((_CHUNK, _H), f32)
    ones128 = jnp.ones((_CHUNK, _H), f32)

    def lin(q):
        return q["w"].T, q["b"].reshape(1, -1)

    ne1t, ne1b = lin(p["node_enc"][0])
    ne2t, ne2b = lin(p["node_enc"][1])
    neg = p["node_enc_ln"]["g"].reshape(1, -1)
    neb = p["node_enc_ln"]["b"].reshape(1, -1)
    ee1t, ee1b = lin(p["edge_enc"][0])
    ee2t, ee2b = lin(p["edge_enc"][1])
    eeg = p["edge_enc_ln"]["g"].reshape(1, -1)
    eeb = p["edge_enc_ln"]["b"].reshape(1, -1)

    emp = []
    for i in range(2):
        w1t, b1 = lin(p["edge_mp"][i][0])
        w2t, b2 = lin(p["edge_mp"][i][1])
        emp.append(dict(
            wst=w1t[0:_H], wdt=w1t[_H:2 * _H], wet=w1t[2 * _H:3 * _H],
            b1=b1, w2t=w2t, b2=b2,
            g=p["edge_mp_ln"][i]["g"].reshape(1, -1),
            bl=p["edge_mp_ln"][i]["b"].reshape(1, -1)))

    nmp = []
    for i in range(2):
        w1t, b1 = lin(p["node_mp"][i][0])
        w2t, b2 = lin(p["node_mp"][i][1])
        nmp.append(dict(
            wht=w1t[0:_H], wat=w1t[_H:2 * _H],
            b1=b1, w2t=w2t, b2=b2,
            g=p["node_mp_ln"][i]["g"].reshape(1, -1),
            bl=p["node_mp_ln"][i]["b"].reshape(1, -1)))

    d1t, db1 = lin(p["node_dec"][0])
    d2t, db2 = lin(p["node_dec"][1])

    h, u, v = _run_node_enc(
        x, [ne1t, ne1b, ne2t, ne2b, neg, neb, emp[0]["wst"], emp[0]["wdt"]])
    e = _run_edge_enc(edge_attr, [ee1t, ee1b, ee2t, ee2b, eeg, eeb])

    cflat = _sc_counts(dst2, ones128)
    c = cflat.reshape(_NC, _NPAD, _H)[:, :_N]

    gu, gv = _sc_gather(u, v, src2, dst2.reshape(1, _E))
    m = emp[0]
    e = _run_edge_mp(gu, gv, e, [m["wet"], m["b1"], m["w2t"], m["b2"],
                                 m["g"], m["bl"]])
    sflat = _sc_segsum(e, dst2, z128)
    s = sflat.reshape(_NC, _NPAD, _H)[:, :_N]
    n = nmp[0]
    h, u, v = _run_node_mp_mid(
        h, s, c, [n["wht"], n["wat"], n["b1"], n["w2t"], n["b2"],
                  n["g"], n["bl"], emp[1]["wst"], emp[1]["wdt"]])

    gu, gv = _sc_gather(u, v, src2, dst2.reshape(1, _E))
    m = emp[1]
    e = _run_edge_mp(gu, gv, e, [m["wet"], m["b1"], m["w2t"], m["b2"],
                                 m["g"], m["bl"]])
    sflat = _sc_segsum(e, dst2, z128)
    s = sflat.reshape(_NC, _NPAD, _H)[:, :_N]
    n = nmp[1]
    out = _run_node_mp_dec(
        h, s, c, [n["wht"], n["wat"], n["b1"], n["w2t"], n["b2"],
                  n["g"], n["bl"], d1t, db1, d2t, db2])
    return out
